# Initial kernel scaffold; baseline (speedup 1.0000x reference)
import jax
import jax.numpy as jnp
from jax.experimental import pallas as pl


def kernel(x, lin_w, lin_b, emb):
    raise NotImplementedError("write your pallas kernel here")


# speedup vs baseline: 5.5505x; 5.5505x over previous
import functools

import jax
import jax.numpy as jnp
from jax import lax
from jax.experimental import pallas as pl
from jax.experimental.pallas import tpu as pltpu
from jax.experimental.pallas import tpu_sc as plsc

B = 16384
N_ATTR = 26
H = 128
ENUM_SIZE = 1001
N_NUM = 13
N_STR = 13
EPS = 1e-05


_NUM_BLK = 2048


def _num_body(xn_ref, w_ref, b_ref, o_ref):
    xn = xn_ref[...]
    j = lax.broadcasted_iota(jnp.int32, (1, N_NUM), 1).astype(jnp.float32)
    mean = 0.2 * j
    scale = 1.0 / (1.0 + 0.1 * j + EPS)
    s = jax.nn.sigmoid((xn - mean) * scale)
    acc = jnp.dot(s, w_ref[...], preferred_element_type=jnp.float32)
    bias = jnp.sum(b_ref[...], axis=0, keepdims=True)
    o_ref[...] = acc + bias


def _numeric_part(xnum, lin_w, lin_b):
    grid = B // _NUM_BLK
    return pl.pallas_call(
        _num_body,
        grid=(grid,),
        in_specs=[
            pl.BlockSpec((_NUM_BLK, N_NUM), lambda i: (i, 0)),
            pl.BlockSpec((N_NUM, H), lambda i: (0, 0)),
            pl.BlockSpec((N_NUM, H), lambda i: (0, 0)),
        ],
        out_specs=pl.BlockSpec((_NUM_BLK, H), lambda i: (i, 0)),
        out_shape=jax.ShapeDtypeStruct((B, H), jnp.float32),
    )(xnum, lin_w, lin_b)


_NC = 2
_NS = 16
_NW = _NC * _NS
_CHUNK = 64
_BPW = B // _NW
_NCHUNK = _BPW // _CHUNK


def _sc_body(table_hbm, xcat_hbm, num_hbm, out_hbm, idx_v, stage_v, acc_v, sem):
    wid = lax.axis_index("s") * _NC + lax.axis_index("c")
    base0 = wid * _BPW
    for t in range(_NCHUNK):
        base = base0 + t * _CHUNK
        idx_copies = [
            pltpu.make_async_copy(
                xcat_hbm.at[pl.ds(j * B + base, _CHUNK)], idx_v.at[j], sem
            )
            for j in range(N_STR)
        ]
        for c in idx_copies:
            c.start()
        for c in idx_copies:
            c.wait()
        for j in range(1, N_STR):
            for v in range(_CHUNK // 16):
                sl = pl.ds(v * 16, 16)
                idx_v[j, sl] = idx_v[j, sl] + (j * ENUM_SIZE)
        copies = [pltpu.make_async_copy(num_hbm.at[pl.ds(base, _CHUNK)], acc_v, sem)]
        for j in range(N_STR):
            copies.append(
                pltpu.make_async_copy(table_hbm.at[idx_v.at[j]], stage_v.at[j], sem)
            )
        for c in copies:
            c.start()
        for c in copies:
            c.wait()

        def _acc_row(r, _):
            for v in range(H // 16):
                sl = pl.ds(v * 16, 16)
                a = acc_v[r, sl]
                for j in range(N_STR):
                    a = a + stage_v[j, r, sl]
                acc_v[r, sl] = a
            return _

        lax.fori_loop(0, _CHUNK, _acc_row, 0)
        pltpu.sync_copy(acc_v, out_hbm.at[pl.ds(base, _CHUNK)])


def _gather_sum(table, xcat_t, num):
    mesh = plsc.VectorSubcoreMesh(core_axis_name="c", subcore_axis_name="s")
    f = pl.kernel(
        _sc_body,
        out_type=jax.ShapeDtypeStruct((B, H), jnp.float32),
        mesh=mesh,
        scratch_types=[
            pltpu.VMEM((N_STR, _CHUNK), jnp.int32),
            pltpu.VMEM((N_STR, _CHUNK, H), jnp.float32),
            pltpu.VMEM((_CHUNK, H), jnp.float32),
            pltpu.SemaphoreType.DMA,
        ],
    )
    return f(table, xcat_t, num)


def kernel(x, lin_w, lin_b, emb):
    xnum = x[:, 0::2]
    xcat_t = x[:, 1::2].astype(jnp.int32).T.reshape(-1)
    table = emb.reshape(N_STR * ENUM_SIZE, H)
    num = _numeric_part(xnum, lin_w, lin_b)
    return _gather_sum(table, xcat_t, num)


# speedup vs baseline: 7.4568x; 1.3434x over previous
import functools

import jax
import jax.numpy as jnp
from jax import lax
from jax.experimental import pallas as pl
from jax.experimental.pallas import tpu as pltpu
from jax.experimental.pallas import tpu_sc as plsc

B = 16384
N_ATTR = 26
H = 128
ENUM_SIZE = 1001
N_NUM = 13
N_STR = 13
EPS = 1e-05


_NUM_BLK = 2048


def _num_body(xn_ref, w_ref, b_ref, o_ref):
    xn = xn_ref[...]
    j = lax.broadcasted_iota(jnp.int32, (1, N_NUM), 1).astype(jnp.float32)
    mean = 0.2 * j
    scale = 1.0 / (1.0 + 0.1 * j + EPS)
    s = jax.nn.sigmoid((xn - mean) * scale)
    acc = jnp.dot(s, w_ref[...], preferred_element_type=jnp.float32)
    bias = jnp.sum(b_ref[...], axis=0, keepdims=True)
    o_ref[...] = acc + bias


def _numeric_part(xnum, lin_w, lin_b):
    grid = B // _NUM_BLK
    return pl.pallas_call(
        _num_body,
        grid=(grid,),
        in_specs=[
            pl.BlockSpec((_NUM_BLK, N_NUM), lambda i: (i, 0)),
            pl.BlockSpec((N_NUM, H), lambda i: (0, 0)),
            pl.BlockSpec((N_NUM, H), lambda i: (0, 0)),
        ],
        out_specs=pl.BlockSpec((_NUM_BLK, H), lambda i: (i, 0)),
        out_shape=jax.ShapeDtypeStruct((B, H), jnp.float32),
    )(xnum, lin_w, lin_b)


_NC = 2
_NS = 16
_NW = _NC * _NS
_CHUNK = 32
_BPW = B // _NW
_NCHUNK = _BPW // _CHUNK
_XPIECE = 128
_NPIECE = _BPW // _XPIECE
_L = 16


def _sc_body(x_hbm, table_hbm, num_hbm, out_hbm,
             idxbuf, stage, accb, semg, semo):
    wid = lax.axis_index("s") * _NC + lax.axis_index("c")
    base0 = wid * _BPW

    idx_copies = [
        pltpu.make_async_copy(
            x_hbm.at[pl.ds(j * B + base0, _BPW)],
            idxbuf.at[pl.ds(j * _BPW, _BPW)], semo[0])
        for j in range(N_STR)
    ]
    for c in idx_copies:
        c.start()
    for c in idx_copies:
        c.wait()

    def fire(t):
        b = t % 2
        base = base0 + t * _CHUNK
        cs = [pltpu.make_async_copy(
            num_hbm.at[pl.ds(base, _CHUNK)], accb.at[b], semg[b])]
        for j in range(N_STR):
            cs.append(pltpu.make_async_copy(
                table_hbm.at[idxbuf.at[pl.ds(j * _BPW + t * _CHUNK, _CHUNK)]],
                stage.at[b, j], semg[b]))
        for c in cs:
            c.start()
        return cs

    inflight = {0: fire(0)}
    out_copies = {}
    for t in range(_NCHUNK):
        b = t % 2
        if t + 1 < _NCHUNK:
            if t - 1 in out_copies:
                out_copies[t - 1].wait()
            inflight[t + 1] = fire(t + 1)
        for c in inflight.pop(t):
            c.wait()

        def _acc_row(r, _):
            for v in range(H // _L):
                sl = pl.ds(v * _L, _L)
                a = accb[b, r, sl]
                for j in range(N_STR):
                    a = a + stage[b, j, r, sl]
                accb[b, r, sl] = a
            return _

        lax.fori_loop(0, _CHUNK, _acc_row, 0)
        oc = pltpu.make_async_copy(
            accb.at[b], out_hbm.at[pl.ds(base0 + t * _CHUNK, _CHUNK)], semo[b])
        oc.start()
        out_copies[t] = oc
    out_copies[_NCHUNK - 2].wait()
    out_copies[_NCHUNK - 1].wait()


def _gather_sum(x, table, num):
    mesh = plsc.VectorSubcoreMesh(core_axis_name="c", subcore_axis_name="s")
    f = pl.kernel(
        _sc_body,
        out_type=jax.ShapeDtypeStruct((B, H), jnp.float32),
        mesh=mesh,
        scratch_types=[
            pltpu.VMEM((N_STR * _BPW,), jnp.int32),
            pltpu.VMEM((2, N_STR, _CHUNK, H), jnp.float32),
            pltpu.VMEM((2, _CHUNK, H), jnp.float32),
            [pltpu.SemaphoreType.DMA, pltpu.SemaphoreType.DMA],
            [pltpu.SemaphoreType.DMA, pltpu.SemaphoreType.DMA],
        ],
    )
    return f(x, table, num)


def kernel(x, lin_w, lin_b, emb):
    xnum = x[:, 0::2]
    offs = (jnp.arange(N_STR, dtype=jnp.int32) * ENUM_SIZE)[:, None]
    xcat_flat = (x[:, 1::2].astype(jnp.int32).T + offs).reshape(-1)
    table = emb.reshape(N_STR * ENUM_SIZE, H)
    num = _numeric_part(xnum, lin_w, lin_b)
    return _gather_sum(xcat_flat, table, num)
